# baseline (speedup 1.0000x reference)
import functools

import jax
import jax.numpy as jnp
from jax.experimental import pallas as pl
from jax.experimental.pallas import tpu as pltpu

N_BINS = 15
_BIN_PAD = 16
_LANES = 128
_WIDE = N_BINS * _LANES


def _ece_kernel(logits_ref, labels_ref, out_ref,
                cnt_ref, cfs_ref, maxlab_ref,
                *, n_total):
    step = pl.program_id(0)
    nsteps = pl.num_programs(0)

    @pl.when(step == 0)
    def _init():
        cnt_ref[...] = jnp.zeros((8, 2 * _WIDE), jnp.float32)
        cfs_ref[...] = jnp.zeros((8, _WIDE), jnp.float32)
        maxlab_ref[0] = 0

    x = logits_ref[...]
    bn, c = x.shape
    m = jnp.max(x, axis=1, keepdims=True)
    e = jnp.exp(x - m)
    ones_j = jnp.ones((c, 128), jnp.float32)
    s_full = jax.lax.dot_general(
        e, ones_j, dimension_numbers=(((1,), (0,)), ((), ())),
        preferred_element_type=jnp.float32)
    conf = e * (1.0 / s_full[:, 0:c])

    idx = jnp.ceil(conf * jnp.float32(N_BINS)) - 1.0
    idx = jnp.where(conf > 0.0, idx, -1.0)

    pad_cfg = ((0, 0, 0), (0, _LANES - c, 0))
    idx_p = jax.lax.pad(idx, jnp.float32(-1.0), pad_cfg)
    conf_p = jax.lax.pad(conf, jnp.float32(0.0), pad_cfg)

    lbl = labels_ref[...]
    maxlab_ref[0] = jnp.maximum(maxlab_ref[0], jnp.max(lbl))
    lbl_bf = lbl.astype(jnp.bfloat16)
    cls_iota_bf = jax.lax.broadcasted_iota(
        jnp.int32, (bn, _LANES), 1).astype(jnp.bfloat16)
    one_bf = jnp.bfloat16(1.0)
    zero_bf = jnp.bfloat16(0.0)
    onehot_bf = jnp.where(lbl_bf == cls_iota_bf, one_bf, zero_bf)

    cnt_chunks = []
    corr_chunks = []
    cfs_chunks = []
    idx_bf = idx_p.astype(jnp.bfloat16)
    for b in range(N_BINS):
        eq_bf = idx_bf == jnp.bfloat16(b)
        m01 = jnp.where(eq_bf, one_bf, zero_bf)
        cnt_chunks.append(m01)
        corr_chunks.append(m01 * onehot_bf)
        eq = idx_p == jnp.float32(b)
        cfs_chunks.append(jnp.where(eq, conf_p, 0.0))
    cnt_wide = jnp.concatenate(cnt_chunks + corr_chunks, axis=1)
    cfs_wide = jnp.concatenate(cfs_chunks, axis=1)
    ones = jnp.ones((1, bn), jnp.float32)
    dn = (((1,), (0,)), ((), ()))
    cnt_row = jax.lax.dot_general(ones.astype(jnp.bfloat16), cnt_wide,
                                  dimension_numbers=dn,
                                  preferred_element_type=jnp.float32)
    cfs_row = jax.lax.dot_general(ones, cfs_wide, dimension_numbers=dn,
                                  preferred_element_type=jnp.float32)
    cnt_ref[0:1, :] += cnt_row
    cfs_ref[0:1, :] += cfs_row

    @pl.when(step == nsteps - 1)
    def _finalize():
        count = cnt_ref[0:1, 0:_WIDE].reshape(N_BINS, _LANES)
        corr = cnt_ref[0:1, _WIDE:2 * _WIDE].reshape(N_BINS, _LANES)
        confsum = cfs_ref[0:1, :].reshape(N_BINS, _LANES)
        num_classes = (maxlab_ref[0] + 1).astype(jnp.float32)
        prop = count * jnp.float32(1.0 / n_total)
        safe = jnp.maximum(count, 1.0)
        acc_in_bin = corr / safe
        avg_conf = confsum / safe
        term = jnp.where(count > 0.0,
                         jnp.abs(avg_conf - acc_in_bin) * prop, 0.0)
        class_sce = jnp.sum(term, axis=0, keepdims=True)
        cls = jax.lax.broadcasted_iota(jnp.int32, (1, _LANES), 1)
        mask = (cls < (maxlab_ref[0] + 1)).astype(jnp.float32)
        out_ref[...] = jnp.sum(class_sce * mask, keepdims=True) / num_classes


def kernel(logits, labels):
    n, c = logits.shape
    bn = n
    for cand in (2000, 1250, 1000, 625, 500, 400, 250, 200, 125, 100):
        if n % cand == 0 and cand % 8 == 0:
            bn = cand
            break
    grid = n // bn
    out = pl.pallas_call(
        functools.partial(_ece_kernel, n_total=n),
        grid=(grid,),
        in_specs=[
            pl.BlockSpec((bn, c), lambda i: (i, 0)),
            pl.BlockSpec((bn, 1), lambda i: (i, 0)),
        ],
        out_specs=pl.BlockSpec((1, 1), lambda i: (0, 0)),
        out_shape=jax.ShapeDtypeStruct((1, 1), jnp.float32),
        scratch_shapes=[
            pltpu.VMEM((8, 2 * _WIDE), jnp.float32),
            pltpu.VMEM((8, _WIDE), jnp.float32),
            pltpu.SMEM((1,), jnp.int32),
        ],
        compiler_params=pltpu.CompilerParams(
            dimension_semantics=("arbitrary",)),
    )(logits, labels.reshape(n, 1))
    return out.reshape(())


# speedup vs baseline: 1.0191x; 1.0191x over previous
import functools

import jax
import jax.numpy as jnp
from jax.experimental import pallas as pl
from jax.experimental.pallas import tpu as pltpu

N_BINS = 15
_BIN_PAD = 16
_LANES = 128
_WIDE = N_BINS * _LANES


def _ece_kernel(logits_ref, labels_ref, out_ref,
                cnt_ref, cfs_ref, corr_ref, maxlab_ref,
                *, n_total):
    step = pl.program_id(0)
    nsteps = pl.num_programs(0)

    @pl.when(step == 0)
    def _init():
        cnt_ref[...] = jnp.zeros((8, _WIDE), jnp.float32)
        cfs_ref[...] = jnp.zeros((8, _WIDE), jnp.float32)
        corr_ref[...] = jnp.zeros((_BIN_PAD, _LANES), jnp.float32)
        maxlab_ref[0] = 0

    x = logits_ref[...]
    bn, c = x.shape
    m = jnp.max(x, axis=1, keepdims=True)
    e = jnp.exp(x - m)
    s = jnp.sum(e, axis=1, keepdims=True)
    conf = e * (1.0 / s)

    idx = jnp.ceil(conf * jnp.float32(N_BINS)) - 1.0
    idx = jnp.where(conf > 0.0, idx, -1.0)

    pad_cfg = ((0, 0, 0), (0, _LANES - c, 0))
    idx_p = jax.lax.pad(idx, jnp.float32(-1.0), pad_cfg)
    conf_p = jax.lax.pad(conf, jnp.float32(0.0), pad_cfg)

    lbl = labels_ref[...]
    maxlab_ref[0] = jnp.maximum(maxlab_ref[0], jnp.max(lbl))
    cls_iota = jax.lax.broadcasted_iota(jnp.int32, (bn, c), 1)
    onehot = (lbl == cls_iota).astype(jnp.float32)

    conf_label = jnp.sum(conf * onehot, axis=1, keepdims=True)
    idx_lab = jnp.ceil(conf_label * jnp.float32(N_BINS)) - 1.0
    idx_lab = jnp.where(conf_label > 0.0, idx_lab, -1.0)
    bin_iota = jax.lax.broadcasted_iota(jnp.int32, (bn, _BIN_PAD), 1)
    a = (bin_iota == idx_lab.astype(jnp.int32)).astype(jnp.bfloat16)
    corr_part = jax.lax.dot_general(
        a, onehot.astype(jnp.bfloat16),
        dimension_numbers=(((0,), (0,)), ((), ())),
        preferred_element_type=jnp.float32)
    corr_ref[:, 0:c] += corr_part

    cnt_chunks = []
    cfs_chunks = []
    one_bf = jnp.bfloat16(1.0)
    zero_bf = jnp.bfloat16(0.0)
    idx_bf = idx_p.astype(jnp.bfloat16)
    for b in range(N_BINS):
        eq_bf = idx_bf == jnp.bfloat16(b)
        cnt_chunks.append(jnp.where(eq_bf, one_bf, zero_bf))
        eq = idx_p == jnp.float32(b)
        cfs_chunks.append(jnp.where(eq, conf_p, 0.0))
    cnt_wide = jnp.concatenate(cnt_chunks, axis=1)
    cfs_wide = jnp.concatenate(cfs_chunks, axis=1)
    ones = jnp.ones((1, bn), jnp.float32)
    dn = (((1,), (0,)), ((), ()))
    cnt_row = jax.lax.dot_general(ones.astype(jnp.bfloat16), cnt_wide,
                                  dimension_numbers=dn,
                                  preferred_element_type=jnp.float32)
    cfs_row = jax.lax.dot_general(ones, cfs_wide, dimension_numbers=dn,
                                  preferred_element_type=jnp.float32)
    cnt_ref[0:1, :] += cnt_row
    cfs_ref[0:1, :] += cfs_row

    @pl.when(step == nsteps - 1)
    def _finalize():
        count = cnt_ref[0:1, :].reshape(N_BINS, _LANES)
        confsum = cfs_ref[0:1, :].reshape(N_BINS, _LANES)
        corr = corr_ref[0:N_BINS, :]
        num_classes = (maxlab_ref[0] + 1).astype(jnp.float32)
        prop = count * jnp.float32(1.0 / n_total)
        safe = jnp.maximum(count, 1.0)
        acc_in_bin = corr / safe
        avg_conf = confsum / safe
        term = jnp.where(count > 0.0,
                         jnp.abs(avg_conf - acc_in_bin) * prop, 0.0)
        class_sce = jnp.sum(term, axis=0, keepdims=True)
        cls = jax.lax.broadcasted_iota(jnp.int32, (1, _LANES), 1)
        mask = (cls < (maxlab_ref[0] + 1)).astype(jnp.float32)
        out_ref[...] = jnp.sum(class_sce * mask, keepdims=True) / num_classes


def kernel(logits, labels):
    n, c = logits.shape
    bn = n
    for cand in (2000, 1250, 1000, 625, 500, 400, 250, 200, 125, 100):
        if n % cand == 0 and cand % 8 == 0:
            bn = cand
            break
    grid = n // bn
    out = pl.pallas_call(
        functools.partial(_ece_kernel, n_total=n),
        grid=(grid,),
        in_specs=[
            pl.BlockSpec((bn, c), lambda i: (i, 0)),
            pl.BlockSpec((bn, 1), lambda i: (i, 0)),
        ],
        out_specs=pl.BlockSpec((1, 1), lambda i: (0, 0)),
        out_shape=jax.ShapeDtypeStruct((1, 1), jnp.float32),
        scratch_shapes=[
            pltpu.VMEM((8, _WIDE), jnp.float32),
            pltpu.VMEM((8, _WIDE), jnp.float32),
            pltpu.VMEM((_BIN_PAD, _LANES), jnp.float32),
            pltpu.SMEM((1,), jnp.int32),
        ],
        compiler_params=pltpu.CompilerParams(
            dimension_semantics=("arbitrary",)),
    )(logits, labels.reshape(n, 1))
    return out.reshape(())
